# Initial kernel scaffold; baseline (speedup 1.0000x reference)
import jax
import jax.numpy as jnp
from jax.experimental import pallas as pl


def kernel(feature, edge_index, W, b):
    raise NotImplementedError("write your pallas kernel here")


# speedup vs baseline: 8.4108x; 8.4108x over previous
import functools

import jax
import jax.numpy as jnp
from jax import lax
from jax.experimental import pallas as pl
from jax.experimental.pallas import tpu as pltpu
from jax.experimental.pallas import tpu_sc as plsc

N_NODES = 10000
N_EDGES = 320000
F = 128

NC = 2
NS = 16
NW = NC * NS

EDGES_PER_TILE = N_EDGES // NW
K = 125
CHUNKS = EDGES_PER_TILE // K
ROWS_PER_TILE = N_NODES // NS
ZROWS = 25

_mesh = plsc.VectorSubcoreMesh(core_axis_name="c", subcore_axis_name="s")


@functools.partial(
    pl.kernel,
    out_type=jax.ShapeDtypeStruct((NC, NS, ROWS_PER_TILE, F), jnp.float32),
    mesh=_mesh,
    scratch_types=[
        pltpu.VMEM((CHUNKS, K), jnp.int32),
        pltpu.VMEM((CHUNKS, K), jnp.int32),
        pltpu.VMEM((K, F), jnp.float32),
        pltpu.VMEM((ZROWS, F), jnp.float32),
        pltpu.VMEM_SHARED((N_NODES, F), jnp.float32),
        pltpu.SemaphoreType.DMA,
    ],
)
def _sc_gather_scatter(feat_hbm, src_hbm, dst_hbm, out_hbm,
                       src_v, dst_v, rows_v, zero_v, accum_sh, sem):
    c = lax.axis_index("c")
    s = lax.axis_index("s")
    wid = s * NC + c

    pltpu.sync_copy(src_hbm.at[wid], src_v)
    pltpu.sync_copy(dst_hbm.at[wid], dst_v)

    def _zstore(q, carry):
        i = q // (F // 16)
        l = q % (F // 16)
        zero_v[i, pl.ds(l * 16, 16)] = jnp.zeros((16,), jnp.float32)
        return carry

    lax.fori_loop(0, ZROWS * (F // 16), _zstore, 0)

    def _zcopy(t, carry):
        pltpu.sync_copy(zero_v,
                        accum_sh.at[pl.ds(s * ROWS_PER_TILE + t * ZROWS, ZROWS)])
        return carry

    lax.fori_loop(0, ROWS_PER_TILE // ZROWS, _zcopy, 0)
    plsc.subcore_barrier()

    def _chunk(j, carry):
        pltpu.async_copy(feat_hbm.at[src_v.at[j]], rows_v, sem).wait()
        pltpu.sync_copy(rows_v, accum_sh.at[dst_v.at[j]], add=True)
        return carry

    lax.fori_loop(0, CHUNKS, _chunk, 0)
    plsc.subcore_barrier()

    pltpu.sync_copy(accum_sh.at[pl.ds(s * ROWS_PER_TILE, ROWS_PER_TILE)],
                    out_hbm.at[c, s])


def _tc_linear_body(p_ref, w_ref, b_ref, o_ref):
    h = p_ref[0] + p_ref[1]
    o_ref[...] = lax.dot_general(
        h, w_ref[...], (((1,), (1,)), ((), ())),
        preferred_element_type=jnp.float32) + b_ref[...]


_BM = 1000


@jax.jit
def _tc_linear(partials, W, b2d):
    return pl.pallas_call(
        _tc_linear_body,
        grid=(N_NODES // _BM,),
        in_specs=[
            pl.BlockSpec((NC, _BM, F), lambda i: (0, i, 0)),
            pl.BlockSpec((F, F), lambda i: (0, 0)),
            pl.BlockSpec((1, F), lambda i: (0, 0)),
        ],
        out_specs=pl.BlockSpec((_BM, F), lambda i: (i, 0)),
        out_shape=jax.ShapeDtypeStruct((N_NODES, F), jnp.float32),
    )(partials, W, b2d)


def kernel(feature, edge_index, W, b):
    src = edge_index[0].astype(jnp.int32).reshape(NW, CHUNKS, K)
    dst = edge_index[1].astype(jnp.int32).reshape(NW, CHUNKS, K)
    partials = _sc_gather_scatter(feature, src, dst)
    partials = partials.reshape(NC, N_NODES, F)
    return _tc_linear(partials, W, b.reshape(1, F))


# speedup vs baseline: 9.8332x; 1.1691x over previous
import functools

import jax
import jax.numpy as jnp
from jax import lax
from jax.experimental import pallas as pl
from jax.experimental.pallas import tpu as pltpu
from jax.experimental.pallas import tpu_sc as plsc

N_NODES = 10000
N_EDGES = 320000
F = 128

NC = 2
NS = 16
FH = F // NC

EDGES_PER_TILE = N_EDGES // NS
K = 125
CHUNKS = EDGES_PER_TILE // K
ROWS_PER_TILE = N_NODES // NS
ZROWS = 25

_mesh = plsc.VectorSubcoreMesh(core_axis_name="c", subcore_axis_name="s")


@functools.partial(
    pl.kernel,
    out_type=jax.ShapeDtypeStruct((NC, NS, ROWS_PER_TILE, FH), jnp.float32),
    mesh=_mesh,
    scratch_types=[
        pltpu.VMEM((CHUNKS, K), jnp.int32),
        pltpu.VMEM((CHUNKS, K), jnp.int32),
        pltpu.VMEM((K, FH), jnp.float32),
        pltpu.VMEM((K, FH), jnp.float32),
        pltpu.VMEM((ZROWS, FH), jnp.float32),
        pltpu.VMEM_SHARED((N_NODES, FH), jnp.float32),
        pltpu.SemaphoreType.DMA,
        pltpu.SemaphoreType.DMA,
    ],
    compiler_params=pltpu.CompilerParams(use_tc_tiling_on_sc=False),
)
def _sc_gather_scatter(feat_hbm, src_hbm, dst_hbm, out_hbm,
                       src_v, dst_v, rows_a, rows_b, zero_v, accum_sh,
                       sem_a, sem_b):
    c = lax.axis_index("c")
    s = lax.axis_index("s")
    feat_h = feat_hbm.at[c]

    pltpu.sync_copy(src_hbm.at[s], src_v)
    pltpu.sync_copy(dst_hbm.at[s], dst_v)

    def _zstore(q, carry):
        i = q // (FH // 16)
        l = q % (FH // 16)
        zero_v[i, pl.ds(l * 16, 16)] = jnp.zeros((16,), jnp.float32)
        return carry

    lax.fori_loop(0, ZROWS * (FH // 16), _zstore, 0)

    def _zcopy(t, carry):
        pltpu.sync_copy(zero_v,
                        accum_sh.at[pl.ds(s * ROWS_PER_TILE + t * ZROWS, ZROWS)])
        return carry

    lax.fori_loop(0, ROWS_PER_TILE // ZROWS, _zcopy, 0)
    plsc.subcore_barrier()

    pltpu.async_copy(feat_h.at[src_v.at[0]], rows_a, sem_a)
    pltpu.async_copy(feat_h.at[src_v.at[1]], rows_b, sem_b)

    def _pair(jj, carry):
        j = 2 * jj
        pltpu.make_async_copy(feat_h.at[src_v.at[j]], rows_a, sem_a).wait()
        pltpu.sync_copy(rows_a, accum_sh.at[dst_v.at[j]], add=True)
        pltpu.async_copy(feat_h.at[src_v.at[j + 2]], rows_a, sem_a)
        pltpu.make_async_copy(feat_h.at[src_v.at[j + 1]], rows_b, sem_b).wait()
        pltpu.sync_copy(rows_b, accum_sh.at[dst_v.at[j + 1]], add=True)
        pltpu.async_copy(feat_h.at[src_v.at[j + 3]], rows_b, sem_b)
        return carry

    lax.fori_loop(0, CHUNKS // 2 - 1, _pair, 0)
    pltpu.make_async_copy(feat_h.at[src_v.at[CHUNKS - 2]], rows_a, sem_a).wait()
    pltpu.sync_copy(rows_a, accum_sh.at[dst_v.at[CHUNKS - 2]], add=True)
    pltpu.make_async_copy(feat_h.at[src_v.at[CHUNKS - 1]], rows_b, sem_b).wait()
    pltpu.sync_copy(rows_b, accum_sh.at[dst_v.at[CHUNKS - 1]], add=True)
    plsc.subcore_barrier()

    pltpu.sync_copy(accum_sh.at[pl.ds(s * ROWS_PER_TILE, ROWS_PER_TILE)],
                    out_hbm.at[c, s])


def _tc_linear_body(p_ref, w_ref, b_ref, o_ref):
    o_ref[...] = (
        lax.dot_general(p_ref[0], w_ref[0], (((1,), (1,)), ((), ())),
                        preferred_element_type=jnp.float32)
        + lax.dot_general(p_ref[1], w_ref[1], (((1,), (1,)), ((), ())),
                          preferred_element_type=jnp.float32)
        + b_ref[...])


_BM = 1000


@jax.jit
def _tc_linear(halves, Wh, b2d):
    return pl.pallas_call(
        _tc_linear_body,
        grid=(N_NODES // _BM,),
        in_specs=[
            pl.BlockSpec((NC, _BM, FH), lambda i: (0, i, 0)),
            pl.BlockSpec((NC, F, FH), lambda i: (0, 0, 0)),
            pl.BlockSpec((1, F), lambda i: (0, 0)),
        ],
        out_specs=pl.BlockSpec((_BM, F), lambda i: (i, 0)),
        out_shape=jax.ShapeDtypeStruct((N_NODES, F), jnp.float32),
    )(halves, Wh, b2d)


def kernel(feature, edge_index, W, b):
    src = edge_index[0].astype(jnp.int32).reshape(NS, CHUNKS, K)
    dst = edge_index[1].astype(jnp.int32).reshape(NS, CHUNKS, K)
    feat_halves = feature.reshape(N_NODES, NC, FH).transpose(1, 0, 2)
    halves = _sc_gather_scatter(feat_halves, src, dst)
    halves = halves.reshape(NC, N_NODES, FH)
    Wh = W.reshape(F, NC, FH).transpose(1, 0, 2)
    return _tc_linear(halves, Wh, b.reshape(1, F))
